# baseline (speedup 1.0000x reference)
import functools

import jax
import jax.numpy as jnp
from jax import lax
from jax.experimental import pallas as pl
from jax.experimental.pallas import tpu as pltpu
from jax.experimental.pallas import tpu_sc as plsc

N = 10000
E = 320000
H = 128
G = 256
C = 10
L = 3

NC = 2
NS = 16
EDGES_PER_TILE = E // NS
CHUNK = 128
NCH = 160
PADDED = NCH * CHUNK
NBUF = 2
NP = 10240
SCRAP = N
ROWS_PER_TILE = NP // NS


def _seg_sum_sc(x_cat, idx_cat, zeros_n):
    mesh = plsc.VectorSubcoreMesh(core_axis_name="c", subcore_axis_name="s")

    @functools.partial(
        pl.kernel,
        out_type=jax.ShapeDtypeStruct((NC, NP, H), jnp.float32),
        mesh=mesh,
        scratch_types=(
            [pltpu.VMEM((2, CHUNK), jnp.int32)] * NBUF
            + [pltpu.VMEM((CHUNK, H), jnp.float32)] * NBUF
            + [pltpu.VMEM_SHARED((NP, H), jnp.float32)]
            + [pltpu.SemaphoreType.DMA] * (2 * NBUF)
        ),
    )
    def seg_sum_kernel(x_hbm, idx_hbm, zeros_hbm, out_hbm, *rest):
        idx = rest[:NBUF]
        rows = rest[NBUF:2 * NBUF]
        acc_sh = rest[2 * NBUF]
        sem_i = rest[2 * NBUF + 1:2 * NBUF + 1 + NBUF]
        sem_r = rest[2 * NBUF + 1 + NBUF:]
        c = lax.axis_index("c")
        s = lax.axis_index("s")
        r0 = s * ROWS_PER_TILE
        pltpu.sync_copy(zeros_hbm.at[pl.ds(r0, ROWS_PER_TILE)],
                        acc_sh.at[pl.ds(r0, ROWS_PER_TILE)])
        plsc.subcore_barrier()
        for b in range(NBUF):
            pltpu.async_copy(idx_hbm.at[c, s, b], idx[b], sem_i[b])
        pltpu.make_async_copy(idx_hbm.at[c, s, 0], idx[0], sem_i[0]).wait()
        pltpu.async_copy(x_hbm.at[idx[0].at[0]], rows[0], sem_r[0])

        def grp(g, carry):
            for bb in range(NBUF):
                i = g * NBUF + bb
                ob = (bb + 1) % NBUF
                pltpu.make_async_copy(x_hbm.at[idx[bb].at[0]], rows[bb],
                                      sem_r[bb]).wait()
                pltpu.sync_copy(rows[bb], acc_sh.at[idx[bb].at[1]], add=True)

                @pl.when(i + NBUF < NCH)
                def _():
                    pltpu.async_copy(idx_hbm.at[c, s, i + NBUF], idx[bb],
                                     sem_i[bb])

                @pl.when(i + 1 < NCH)
                def _():
                    pltpu.make_async_copy(idx_hbm.at[c, s, 0], idx[ob],
                                          sem_i[ob]).wait()
                    pltpu.async_copy(x_hbm.at[idx[ob].at[0]], rows[ob],
                                     sem_r[ob])
            return carry

        lax.fori_loop(0, NCH // NBUF, grp, 0)
        plsc.subcore_barrier()
        pltpu.sync_copy(acc_sh.at[pl.ds(r0, ROWS_PER_TILE)],
                        out_hbm.at[c, pl.ds(r0, ROWS_PER_TILE)])

    return seg_sum_kernel(x_cat, idx_cat, zeros_n)


def _prep_edges(src, dst):
    src2 = jnp.pad(src.reshape(NS, EDGES_PER_TILE),
                   ((0, 0), (0, PADDED - EDGES_PER_TILE)))
    dst2 = jnp.pad(dst.reshape(NS, EDGES_PER_TILE),
                   ((0, 0), (0, PADDED - EDGES_PER_TILE)),
                   constant_values=SCRAP)
    return jnp.stack([src2.reshape(NS, NCH, CHUNK),
                      dst2.reshape(NS, NCH, CHUNK)], axis=2)


def _bn(x, g, b):
    m = jnp.mean(x, axis=0)
    v = jnp.mean((x - m) ** 2, axis=0)
    return g * (x - m) / jnp.sqrt(v + 1e-5) + b


def _seg_mean(x, ids):
    s = jax.ops.segment_sum(x, ids, num_segments=G)
    c = jax.ops.segment_sum(jnp.ones((x.shape[0], 1), x.dtype), ids, num_segments=G)
    return s / jnp.maximum(c, 1.0)


def _seg_max(x, ids):
    m = jax.ops.segment_max(x, ids, num_segments=G)
    return jnp.where(jnp.isfinite(m), m, 0.0)


def _final_linear_body(r_ref, w_ref, b_ref, o_ref):
    o_ref[...] = r_ref[...] @ w_ref[...] + b_ref[...]


def kernel(x_visit, x_service, edge_index_vs, edge_index_sv, batch_visit, batch_service,
           Wrel_vs, brel_vs, Wroot_vs, Wrel_sv, brel_sv, Wroot_sv,
           bn_g_visit, bn_b_visit, bn_g_service, bn_b_service, lin_W, lin_b):
    idx_cat = jnp.stack([_prep_edges(edge_index_vs[0], edge_index_vs[1]),
                         _prep_edges(edge_index_sv[0] + N, edge_index_sv[1])])
    zeros_n = jnp.zeros((NP, H), jnp.float32)
    xv, xs = x_visit, x_service
    readout = jnp.zeros((G, 2 * H), jnp.float32)
    for l in range(L):
        x_cat = jnp.concatenate([xv, xs], axis=0)
        msg = _seg_sum_sc(x_cat, idx_cat, zeros_n)
        msg_s, msg_v = msg[0, :N], msg[1, :N]
        out_s = msg_s @ Wrel_vs[l] + brel_vs[l] + xs @ Wroot_vs[l]
        out_v = msg_v @ Wrel_sv[l] + brel_sv[l] + xv @ Wroot_sv[l]
        xv = _bn(jax.nn.relu(out_v), bn_g_visit, bn_b_visit)
        xs = _bn(jax.nn.relu(out_s), bn_g_service, bn_b_service)
        mean_pool = _seg_mean(xv, batch_visit) + _seg_mean(xs, batch_service)
        max_pool = _seg_max(xv, batch_visit) + _seg_max(xs, batch_service)
        readout = readout + jnp.concatenate([mean_pool, max_pool], axis=1)
    return pl.pallas_call(
        _final_linear_body,
        out_shape=jax.ShapeDtypeStruct((G, C), jnp.float32),
    )(readout, lin_W, lin_b)


# speedup vs baseline: 1.0709x; 1.0709x over previous
import functools

import jax
import jax.numpy as jnp
from jax import lax
from jax.experimental import pallas as pl
from jax.experimental.pallas import tpu as pltpu
from jax.experimental.pallas import tpu_sc as plsc

N = 10000
E = 320000
H = 128
G = 256
C = 10
L = 3

NC = 2
NS = 16
EDGES_PER_TILE = E // NS
CHUNK = 128
NCH = 160
BLK = 8
NBLK = NCH // BLK
PADDED = NCH * CHUNK
NBUF = 2
NP = 10240
SCRAP = N
ROWS_PER_TILE = NP // NS


def _seg_sum_sc(x_cat, idx_cat, zeros_n):
    mesh = plsc.VectorSubcoreMesh(core_axis_name="c", subcore_axis_name="s")

    @functools.partial(
        pl.kernel,
        out_type=jax.ShapeDtypeStruct((NC, NP, H), jnp.float32),
        mesh=mesh,
        scratch_types=(
            [pltpu.VMEM((BLK, 2, CHUNK), jnp.int32)] * 2
            + [pltpu.VMEM((CHUNK, H), jnp.float32)] * NBUF
            + [pltpu.VMEM_SHARED((NP, H), jnp.float32)]
            + [pltpu.SemaphoreType.DMA] * 4
        ),
    )
    def seg_sum_kernel(x_hbm, idx_hbm, zeros_hbm, out_hbm,
                       ib0, ib1, row0, row1, acc_sh, si0, si1, sr0, sr1):
        ib = (ib0, ib1)
        rows = (row0, row1)
        sem_i = (si0, si1)
        sem_r = (sr0, sr1)
        c = lax.axis_index("c")
        s = lax.axis_index("s")
        r0 = s * ROWS_PER_TILE
        pltpu.sync_copy(zeros_hbm.at[pl.ds(r0, ROWS_PER_TILE)],
                        acc_sh.at[pl.ds(r0, ROWS_PER_TILE)])
        pltpu.sync_copy(idx_hbm.at[c, s, 0], ib0)
        pltpu.async_copy(idx_hbm.at[c, s, 1], ib1, si1)
        plsc.subcore_barrier()
        pltpu.async_copy(x_hbm.at[ib0.at[0, 0]], row0, sr0)
        last_g = NBLK // 2 - 1

        def pair(g, carry):
            for j in range(2 * BLK):
                blk, jj = divmod(j, BLK)
                bb = j % 2
                ob = 1 - bb
                pltpu.make_async_copy(x_hbm.at[ib0.at[0, 0]], rows[bb],
                                      sem_r[bb]).wait()
                if j == BLK - 1:
                    pltpu.make_async_copy(idx_hbm.at[c, s, 0], ib1,
                                          sem_i[1]).wait()
                    pltpu.async_copy(x_hbm.at[ib1.at[0, 0]], rows[ob],
                                     sem_r[ob])
                elif j == 2 * BLK - 1:
                    @pl.when(g < last_g)
                    def _():
                        pltpu.make_async_copy(idx_hbm.at[c, s, 0], ib0,
                                              sem_i[0]).wait()
                        pltpu.async_copy(x_hbm.at[ib0.at[0, 0]], rows[ob],
                                         sem_r[ob])
                else:
                    pltpu.async_copy(x_hbm.at[ib[blk].at[jj + 1, 0]],
                                     rows[ob], sem_r[ob])
                pltpu.sync_copy(rows[bb], acc_sh.at[ib[blk].at[jj, 1]],
                                add=True)
                if j == BLK - 1:
                    @pl.when(g < last_g)
                    def _():
                        pltpu.async_copy(idx_hbm.at[c, s, 2 * g + 2], ib0,
                                         sem_i[0])
                elif j == 2 * BLK - 1:
                    @pl.when(g < last_g)
                    def _():
                        pltpu.async_copy(idx_hbm.at[c, s, 2 * g + 3], ib1,
                                         sem_i[1])
            return carry

        lax.fori_loop(0, NBLK // 2, pair, 0)
        plsc.subcore_barrier()
        pltpu.sync_copy(acc_sh.at[pl.ds(r0, ROWS_PER_TILE)],
                        out_hbm.at[c, pl.ds(r0, ROWS_PER_TILE)])

    return seg_sum_kernel(x_cat, idx_cat, zeros_n)


def _prep_edges(src, dst):
    src2 = jnp.pad(src.reshape(NS, EDGES_PER_TILE),
                   ((0, 0), (0, PADDED - EDGES_PER_TILE)))
    dst2 = jnp.pad(dst.reshape(NS, EDGES_PER_TILE),
                   ((0, 0), (0, PADDED - EDGES_PER_TILE)),
                   constant_values=SCRAP)
    both = jnp.stack([src2.reshape(NS, NCH, CHUNK),
                      dst2.reshape(NS, NCH, CHUNK)], axis=2)
    return both.reshape(NS, NBLK, BLK, 2, CHUNK)


def _bn(x, g, b):
    m = jnp.mean(x, axis=0)
    v = jnp.mean((x - m) ** 2, axis=0)
    return g * (x - m) / jnp.sqrt(v + 1e-5) + b


def _seg_mean(x, ids):
    s = jax.ops.segment_sum(x, ids, num_segments=G)
    c = jax.ops.segment_sum(jnp.ones((x.shape[0], 1), x.dtype), ids, num_segments=G)
    return s / jnp.maximum(c, 1.0)


def _seg_max(x, ids):
    m = jax.ops.segment_max(x, ids, num_segments=G)
    return jnp.where(jnp.isfinite(m), m, 0.0)


def _final_linear_body(r_ref, w_ref, b_ref, o_ref):
    o_ref[...] = r_ref[...] @ w_ref[...] + b_ref[...]


def kernel(x_visit, x_service, edge_index_vs, edge_index_sv, batch_visit, batch_service,
           Wrel_vs, brel_vs, Wroot_vs, Wrel_sv, brel_sv, Wroot_sv,
           bn_g_visit, bn_b_visit, bn_g_service, bn_b_service, lin_W, lin_b):
    idx_cat = jnp.stack([_prep_edges(edge_index_vs[0], edge_index_vs[1]),
                         _prep_edges(edge_index_sv[0] + N, edge_index_sv[1])])
    zeros_n = jnp.zeros((NP, H), jnp.float32)
    xv, xs = x_visit, x_service
    readout = jnp.zeros((G, 2 * H), jnp.float32)
    for l in range(L):
        x_cat = jnp.concatenate([xv, xs], axis=0)
        msg = _seg_sum_sc(x_cat, idx_cat, zeros_n)
        msg_s, msg_v = msg[0, :N], msg[1, :N]
        out_s = msg_s @ Wrel_vs[l] + brel_vs[l] + xs @ Wroot_vs[l]
        out_v = msg_v @ Wrel_sv[l] + brel_sv[l] + xv @ Wroot_sv[l]
        xv = _bn(jax.nn.relu(out_v), bn_g_visit, bn_b_visit)
        xs = _bn(jax.nn.relu(out_s), bn_g_service, bn_b_service)
        mean_pool = _seg_mean(xv, batch_visit) + _seg_mean(xs, batch_service)
        max_pool = _seg_max(xv, batch_visit) + _seg_max(xs, batch_service)
        readout = readout + jnp.concatenate([mean_pool, max_pool], axis=1)
    return pl.pallas_call(
        _final_linear_body,
        out_shape=jax.ShapeDtypeStruct((G, C), jnp.float32),
    )(readout, lin_W, lin_b)
